# baseline (speedup 1.0000x reference)
import functools

import jax
import jax.numpy as jnp
from jax import lax
from jax.experimental import pallas as pl
from jax.experimental.pallas import tpu as pltpu
from jax.experimental.pallas import tpu_sc as plsc

N = 10000
DEG = 32
E = N * DEG
HD = 128
OD = 128
NL = 3

_NC, _NS = 2, 16
_NW = _NC * _NS
_CN = 4
_CE = _CN * DEG
_NCHUNK = N // _CN
_NCHP = 2504
_GRP = 8
_LANES = 16
_NODES_A = 4800
_NODES_B = N - _NODES_A
_WCH = 48


def _sc_gr_body(cbase, npart, table_hbm, src_hbm, w_hbm, out_hbm, idx_v, w_v,
                rb0, rb1, stage, sem0, sem1):
    wid = lax.axis_index("s") * _NC + lax.axis_index("c")
    c0 = cbase + 8 * ((npart * wid) // (8 * _NW))
    lc0 = c0 - cbase
    pltpu.sync_copy(src_hbm.at[pl.ds(c0, _WCH)], idx_v)
    pltpu.sync_copy(w_hbm.at[pl.ds(c0 * _CN * DEG, _WCH * _CN * DEG)], w_v)

    def start(lc, rb, sem):
        pltpu.async_copy(table_hbm.at[idx_v.at[lc]], rb, sem)

    def wait(lc, rb, sem):
        pltpu.make_async_copy(table_hbm.at[idx_v.at[lc]], rb, sem).wait()

    def compute(lc, srow0, rb):
        def node(j, carry):
            row0 = j * DEG
            wbase = (lc * _CN + j) * DEG
            wv = (w_v[pl.ds(wbase, _LANES)], w_v[pl.ds(wbase + _LANES, _LANES)])
            s = [None] * 8
            m = [None] * 8
            a = [None] * 8
            for k in range(DEG):
                wk = wv[k // _LANES][k % _LANES]
                for d in range(8):
                    v = rb[row0 + k, pl.ds(d * _LANES, _LANES)]
                    if k == 0:
                        s[d] = v
                        m[d] = v
                        a[d] = wk * v
                    else:
                        s[d] = s[d] + v
                        m[d] = jnp.maximum(m[d], v)
                        a[d] = a[d] + wk * v
            for d in range(8):
                sl = pl.ds(d * _LANES, _LANES)
                stage[srow0 + j, 0, sl] = s[d]
                stage[srow0 + j, 1, sl] = m[d]
                stage[srow0 + j, 2, sl] = a[d]
            return carry

        lax.fori_loop(0, _CN, node, 0)

    start(0, rb0, sem0)

    def group(g, carry):
        def pair(p, carry2):
            lcA = g * _GRP + 2 * p
            start(lcA + 1, rb1, sem1)
            wait(lcA, rb0, sem0)
            compute(lcA, 2 * p * _CN, rb0)

            @pl.when(lcA + 2 < _WCH)
            def _():
                start(lcA + 2, rb0, sem0)

            wait(lcA + 1, rb1, sem1)
            compute(lcA + 1, (2 * p + 1) * _CN, rb1)
            return carry2

        lax.fori_loop(0, _GRP // 2, pair, 0)
        pltpu.sync_copy(stage, out_hbm.at[pl.ds((lc0 + g * _GRP) * _CN,
                                                _GRP * _CN)])
        return carry

    lax.fori_loop(0, _WCH // _GRP, group, 0)


def _make_gr(cbase, npart, nout_chunks):
    scratch = [
        pltpu.VMEM((_WCH, _CE), jnp.int32),
        pltpu.VMEM((_WCH * _CN * DEG,), jnp.float32),
        pltpu.VMEM((_CE, HD), jnp.float32),
        pltpu.VMEM((_CE, HD), jnp.float32),
        pltpu.VMEM((_GRP * _CN, 3, HD), jnp.float32),
        pltpu.SemaphoreType.DMA,
        pltpu.SemaphoreType.DMA,
    ]
    return functools.partial(
        pl.kernel,
        out_type=jax.ShapeDtypeStruct((nout_chunks * _CN, 3, HD), jnp.float32),
        mesh=plsc.VectorSubcoreMesh(core_axis_name="c", subcore_axis_name="s"),
        scratch_types=scratch,
    )(functools.partial(_sc_gr_body, cbase, npart))


@functools.cache
def _sc_gr_kernels():
    na = _NODES_A // _CN
    nb = _NODES_B // _CN
    return (_make_gr(0, na, 1208),
            _make_gr(na, nb, _NCHP - na))


_NBE = 1000
_NB = 400


def _enc_body(h_ref, eig_ref, T_ref, ws_ref, wd_ref,
              x_ref, xs_ref, xd_ref, w_ref):
    ids = lax.broadcasted_iota(jnp.int32, (_NBE, 128), 1)
    x = jnp.zeros((_NBE, HD), jnp.float32)
    for f in range(9):
        oh = (ids == h_ref[:, f][:, None]).astype(jnp.float32)
        x = x + jnp.dot(oh, T_ref[f], preferred_element_type=jnp.float32)
    x_ref[...] = x
    xs_ref[...] = jnp.dot(x, ws_ref[...], preferred_element_type=jnp.float32)
    xd_ref[...] = jnp.dot(x, wd_ref[...], preferred_element_type=jnp.float32)
    eig1 = eig_ref[...]
    absum = jnp.sum(jnp.abs(eig1), axis=1, keepdims=True) + 1e-8
    w_ref[...] = eig1 / absum


def _enc_call(h, eig132, T, ws, wd):
    nb = N // _NBE
    return pl.pallas_call(
        _enc_body,
        grid=(nb,),
        in_specs=[
            pl.BlockSpec((_NBE, 128), lambda i: (i, 0)),
            pl.BlockSpec((_NBE, DEG), lambda i: (i, 0)),
            pl.BlockSpec((9, 128, HD), lambda i: (0, 0, 0)),
            pl.BlockSpec((HD, HD), lambda i: (0, 0)),
            pl.BlockSpec((HD, HD), lambda i: (0, 0)),
        ],
        out_specs=[
            pl.BlockSpec((_NBE, HD), lambda i: (i, 0)),
            pl.BlockSpec((_NBE, HD), lambda i: (i, 0)),
            pl.BlockSpec((_NBE, HD), lambda i: (i, 0)),
            pl.BlockSpec((_NBE, DEG), lambda i: (i, 0)),
        ],
        out_shape=[
            jax.ShapeDtypeStruct((N, HD), jnp.float32),
            jax.ShapeDtypeStruct((N, HD), jnp.float32),
            jax.ShapeDtypeStruct((N, HD), jnp.float32),
            jax.ShapeDtypeStruct((N, DEG), jnp.float32),
        ],
    )(h, eig132, T, ws, wd)


def _agg_body(R_ref, w_ref, x_ref, xd_ref, sn_ref, preb_ref,
              p0_ref, p12_ref, p3_ref, p4_ref, pb_ref, postb_ref,
              h2_ref, stats_ref):
    i = pl.program_id(0)
    w = w_ref[...]
    x = x_ref[...]
    base = xd_ref[...] + preb_ref[...]
    gsum = R_ref[:, 0, :]
    gmax = R_ref[:, 1, :]
    gw = R_ref[:, 2, :]
    sw = jnp.sum(w, axis=1, keepdims=True)
    a_dir = jnp.abs(gw + sw * (base - x))
    h2 = (jnp.dot(x, p0_ref[...], preferred_element_type=jnp.float32)
          + jnp.dot(gsum, p12_ref[...], preferred_element_type=jnp.float32)
          + jnp.dot(gmax, p3_ref[...], preferred_element_type=jnp.float32)
          + jnp.dot(a_dir, p4_ref[...], preferred_element_type=jnp.float32)
          + jnp.dot(base, pb_ref[...], preferred_element_type=jnp.float32)
          + postb_ref[...]) * sn_ref[...]
    h2_ref[...] = h2

    @pl.when(i == 0)
    def _():
        stats_ref[...] = jnp.zeros_like(stats_ref)

    s1 = jnp.sum(h2, axis=0, keepdims=True)
    s2 = jnp.sum(h2 * h2, axis=0, keepdims=True)
    stats_ref[...] += jnp.concatenate([s1, s2], axis=0)


def _agg_call(R, w, x, xd, snorm, preb, p0, p12, p3, p4, pb, postb,
              nnodes, node_off):
    nb = nnodes // _NB
    off = node_off // _NB
    blk = lambda c: pl.BlockSpec((_NB, c), lambda i, o=off: (i + o, 0))
    full = lambda r, c: pl.BlockSpec((r, c), lambda i: (0, 0))
    return pl.pallas_call(
        _agg_body,
        grid=(nb,),
        in_specs=[
            pl.BlockSpec((_NB, 3, HD), lambda i: (i, 0, 0)),
            blk(DEG), blk(HD), blk(HD), blk(1),
            full(1, HD), full(HD, OD), full(HD, OD), full(HD, OD), full(HD, OD),
            full(HD, OD), full(1, OD),
        ],
        out_specs=[
            pl.BlockSpec((_NB, OD), lambda i: (i, 0)),
            pl.BlockSpec((2, OD), lambda i: (0, 0)),
        ],
        out_shape=[
            jax.ShapeDtypeStruct((nnodes, OD), jnp.float32),
            jax.ShapeDtypeStruct((2, OD), jnp.float32),
        ],
    )(R, w, x, xd, snorm, preb, p0, p12, p3, p4, pb, postb)


def _bn_scale(stats_ref, g_ref, b_ref):
    st = stats_ref[...]
    mu = st[0:1, :] * (1.0 / N)
    var = st[1:2, :] * (1.0 / N) - mu * mu
    scale = g_ref[...] * lax.rsqrt(var + 1e-5)
    return mu, scale, b_ref[...]


def _bnnext_body(h2_ref, stats_ref, g_ref, b_ref, ws_ref, wd_ref,
                 x_ref, xs_ref, xd_ref):
    mu, scale, b = _bn_scale(stats_ref, g_ref, b_ref)
    xn = jnp.maximum((h2_ref[...] - mu) * scale + b, 0.0)
    x_ref[...] = xn
    xs_ref[...] = jnp.dot(xn, ws_ref[...], preferred_element_type=jnp.float32)
    xd_ref[...] = jnp.dot(xn, wd_ref[...], preferred_element_type=jnp.float32)


def _bnnext_call(h2, stats, g, b, ws, wd):
    nb = N // _NBE
    full = lambda r, c: pl.BlockSpec((r, c), lambda i: (0, 0))
    return pl.pallas_call(
        _bnnext_body,
        grid=(nb,),
        in_specs=[
            pl.BlockSpec((_NBE, OD), lambda i: (i, 0)),
            full(2, OD), full(1, OD), full(1, OD), full(HD, HD), full(HD, HD),
        ],
        out_specs=[
            pl.BlockSpec((_NBE, HD), lambda i: (i, 0)),
            pl.BlockSpec((_NBE, HD), lambda i: (i, 0)),
            pl.BlockSpec((_NBE, HD), lambda i: (i, 0)),
        ],
        out_shape=[
            jax.ShapeDtypeStruct((N, HD), jnp.float32),
            jax.ShapeDtypeStruct((N, HD), jnp.float32),
            jax.ShapeDtypeStruct((N, HD), jnp.float32),
        ],
    )(h2, stats, g, b, ws, wd)


def _bnout_body(h2_ref, stats_ref, g_ref, b_ref,
                r0w_ref, r0b_ref, r1w_ref, r1b_ref, r2w_ref, r2b_ref,
                y_ref, hsum_ref):
    i = pl.program_id(0)
    mu, scale, b = _bn_scale(stats_ref, g_ref, b_ref)
    xn = jnp.maximum((h2_ref[...] - mu) * scale + b, 0.0)

    @pl.when(i == 0)
    def _():
        hsum_ref[...] = jnp.zeros_like(hsum_ref)

    hsum_ref[...] += jnp.sum(xn, axis=0, keepdims=True)

    @pl.when(i == (N // _NBE) - 1)
    def _():
        hg = hsum_ref[...] * (1.0 / N)
        y0 = jnp.maximum(
            jnp.dot(hg, r0w_ref[...], preferred_element_type=jnp.float32)
            + r0b_ref[...], 0.0)
        y1 = jnp.maximum(
            jnp.dot(y0, r1w_ref[...], preferred_element_type=jnp.float32)
            + r1b_ref[...], 0.0)
        y_ref[...] = (jnp.dot(y1, r2w_ref[...], preferred_element_type=jnp.float32)
                      + r2b_ref[...])


def _bnout_call(h2, stats, g, b, r0W, r0b, r1W, r1b, r2W, r2b):
    nb = N // _NBE
    full = lambda r, c: pl.BlockSpec((r, c), lambda i: (0, 0))
    return pl.pallas_call(
        _bnout_body,
        grid=(nb,),
        in_specs=[
            pl.BlockSpec((_NBE, OD), lambda i: (i, 0)),
            full(2, OD), full(1, OD), full(1, OD),
            full(OD, OD // 2), full(1, OD // 2),
            full(OD // 2, OD // 4), full(1, OD // 4),
            full(OD // 4, 128), full(1, 128),
        ],
        out_specs=pl.BlockSpec((1, 128), lambda i: (0, 0)),
        out_shape=jax.ShapeDtypeStruct((1, 128), jnp.float32),
        scratch_shapes=[pltpu.VMEM((1, OD), jnp.float32)],
    )(h2, stats, g, b, r0W, r0b, r1W, r1b, r2W, r2b)


def kernel(h, edge_index, eig, e, snorm_n, atom_emb, pre_W, pre_b, post_W,
           post_b, bn_g, bn_b, r0W, r0b, r1W, r1b, r2W, r2b):
    src = edge_index[0]
    src3 = jnp.pad(src, (0, _NCHP * _CE - E)).reshape(_NCHP, _CE)
    eig132 = eig.T[1].reshape(N, DEG)
    T = jnp.zeros((9, 128, HD), jnp.float32).at[:, :119, :].set(atom_emb)
    Ws = pre_W[:, :HD, :]
    Wd = pre_W[:, HD:, :]
    P0 = post_W[:, 0:HD]
    P12 = post_W[:, HD:2 * HD] / float(DEG) + post_W[:, 2 * HD:3 * HD]
    P3 = post_W[:, 3 * HD:4 * HD]
    P4 = post_W[:, 4 * HD:5 * HD]
    Pb = float(DEG) * P12 + P3
    preb = pre_b.reshape(NL, 1, HD)
    postb = post_b.reshape(NL, 1, OD)
    bng = bn_g.reshape(NL, 1, OD)
    bnb = bn_b.reshape(NL, 1, OD)

    h128 = jnp.pad(h, ((0, 0), (0, 128 - h.shape[1])))
    x, xs, xd, w = _enc_call(h128, eig132, T, Ws[0], Wd[0])
    grA, grB = _sc_gr_kernels()
    y = None
    wpad = jnp.pad(w, ((0, _NCHP * _CN - N), (0, 0))).reshape(-1)
    for l in range(NL):
        RA = grA(xs, src3, wpad)
        RB = grB(xs, src3, wpad)
        h2a, sa = _agg_call(RA, w, x, xd, snorm_n, preb[l],
                            P0[l], P12[l], P3[l], P4[l], Pb[l], postb[l],
                            _NODES_A, 0)
        h2b, sb = _agg_call(RB, w, x, xd, snorm_n, preb[l],
                            P0[l], P12[l], P3[l], P4[l], Pb[l], postb[l],
                            _NODES_B, _NODES_A)
        stats = sa + sb
        h2 = jnp.concatenate([h2a, h2b], axis=0)
        if l < NL - 1:
            x, xs, xd = _bnnext_call(h2, stats, bng[l], bnb[l],
                                     Ws[l + 1], Wd[l + 1])
        else:
            y = _bnout_call(h2, stats, bng[l], bnb[l],
                            r0W, r0b.reshape(1, -1), r1W, r1b.reshape(1, -1),
                            r2W, r2b.reshape(1, -1))
    return y


# speedup vs baseline: 1.0756x; 1.0756x over previous
import functools

import jax
import jax.numpy as jnp
from jax import lax
from jax.experimental import pallas as pl
from jax.experimental.pallas import tpu as pltpu
from jax.experimental.pallas import tpu_sc as plsc

N = 10000
DEG = 32
E = N * DEG
HD = 128
OD = 128
NL = 3

_NC, _NS = 2, 16
_NW = _NC * _NS
_CN = 4
_CE = _CN * DEG
_NCHUNK = N // _CN
_NCHP = 2504
_GRP = 8
_LANES = 16
_WCH = 88


def _sc_gr_body(cbase, npart, table_hbm, src_hbm, w_hbm, out_hbm, idx_v, w_v,
                rb0, rb1, stage, sem0, sem1):
    wid = lax.axis_index("s") * _NC + lax.axis_index("c")
    c0 = cbase + 8 * ((npart * wid) // (8 * _NW))
    lc0 = c0 - cbase
    pltpu.sync_copy(src_hbm.at[pl.ds(c0, _WCH)], idx_v)
    pltpu.sync_copy(w_hbm.at[pl.ds(c0 * _CN * DEG, _WCH * _CN * DEG)], w_v)

    def start(lc, rb, sem):
        pltpu.async_copy(table_hbm.at[idx_v.at[lc]], rb, sem)

    def wait(lc, rb, sem):
        pltpu.make_async_copy(table_hbm.at[idx_v.at[lc]], rb, sem).wait()

    def compute(lc, srow0, rb):
        def node(j, carry):
            row0 = j * DEG
            wbase = (lc * _CN + j) * DEG
            wv = (w_v[pl.ds(wbase, _LANES)], w_v[pl.ds(wbase + _LANES, _LANES)])
            s = [None] * 8
            m = [None] * 8
            a = [None] * 8
            for k in range(DEG):
                wk = wv[k // _LANES][k % _LANES]
                for d in range(8):
                    v = rb[row0 + k, pl.ds(d * _LANES, _LANES)]
                    if k == 0:
                        s[d] = v
                        m[d] = v
                        a[d] = wk * v
                    else:
                        s[d] = s[d] + v
                        m[d] = jnp.maximum(m[d], v)
                        a[d] = a[d] + wk * v
            for d in range(8):
                sl = pl.ds(d * _LANES, _LANES)
                stage[srow0 + j, 0, sl] = s[d]
                stage[srow0 + j, 1, sl] = m[d]
                stage[srow0 + j, 2, sl] = a[d]
            return carry

        lax.fori_loop(0, _CN, node, 0)

    start(0, rb0, sem0)

    def group(g, carry):
        def pair(p, carry2):
            lcA = g * _GRP + 2 * p
            start(lcA + 1, rb1, sem1)
            wait(lcA, rb0, sem0)
            compute(lcA, 2 * p * _CN, rb0)

            @pl.when(lcA + 2 < _WCH)
            def _():
                start(lcA + 2, rb0, sem0)

            wait(lcA + 1, rb1, sem1)
            compute(lcA + 1, (2 * p + 1) * _CN, rb1)
            return carry2

        lax.fori_loop(0, _GRP // 2, pair, 0)
        pltpu.sync_copy(stage, out_hbm.at[pl.ds((lc0 + g * _GRP) * _CN,
                                                _GRP * _CN)])
        return carry

    lax.fori_loop(0, _WCH // _GRP, group, 0)


def _make_gr(cbase, npart, nout_chunks):
    scratch = [
        pltpu.VMEM((_WCH, _CE), jnp.int32),
        pltpu.VMEM((_WCH * _CN * DEG,), jnp.float32),
        pltpu.VMEM((_CE, HD), jnp.float32),
        pltpu.VMEM((_CE, HD), jnp.float32),
        pltpu.VMEM((_GRP * _CN, 3, HD), jnp.float32),
        pltpu.SemaphoreType.DMA,
        pltpu.SemaphoreType.DMA,
    ]
    return functools.partial(
        pl.kernel,
        out_type=jax.ShapeDtypeStruct((nout_chunks * _CN, 3, HD), jnp.float32),
        mesh=plsc.VectorSubcoreMesh(core_axis_name="c", subcore_axis_name="s"),
        scratch_types=scratch,
    )(functools.partial(_sc_gr_body, cbase, npart))


@functools.cache
def _sc_gr_kernel():
    return _make_gr(0, _NCHUNK, _NCHP)


_NBE = 1000
_NB = 400


def _enc_body(h_ref, eig_ref, T_ref, ws_ref, wd_ref,
              x_ref, xs_ref, xd_ref, w_ref):
    ids = lax.broadcasted_iota(jnp.int32, (_NBE, 128), 1)
    x = jnp.zeros((_NBE, HD), jnp.float32)
    for f in range(9):
        oh = (ids == h_ref[:, f][:, None]).astype(jnp.float32)
        x = x + jnp.dot(oh, T_ref[f], preferred_element_type=jnp.float32)
    x_ref[...] = x
    xs_ref[...] = jnp.dot(x, ws_ref[...], preferred_element_type=jnp.float32)
    xd_ref[...] = jnp.dot(x, wd_ref[...], preferred_element_type=jnp.float32)
    eig1 = eig_ref[...]
    absum = jnp.sum(jnp.abs(eig1), axis=1, keepdims=True) + 1e-8
    w_ref[...] = eig1 / absum


def _enc_call(h, eig132, T, ws, wd):
    nb = N // _NBE
    return pl.pallas_call(
        _enc_body,
        grid=(nb,),
        in_specs=[
            pl.BlockSpec((_NBE, 128), lambda i: (i, 0)),
            pl.BlockSpec((_NBE, DEG), lambda i: (i, 0)),
            pl.BlockSpec((9, 128, HD), lambda i: (0, 0, 0)),
            pl.BlockSpec((HD, HD), lambda i: (0, 0)),
            pl.BlockSpec((HD, HD), lambda i: (0, 0)),
        ],
        out_specs=[
            pl.BlockSpec((_NBE, HD), lambda i: (i, 0)),
            pl.BlockSpec((_NBE, HD), lambda i: (i, 0)),
            pl.BlockSpec((_NBE, HD), lambda i: (i, 0)),
            pl.BlockSpec((_NBE, DEG), lambda i: (i, 0)),
        ],
        out_shape=[
            jax.ShapeDtypeStruct((N, HD), jnp.float32),
            jax.ShapeDtypeStruct((N, HD), jnp.float32),
            jax.ShapeDtypeStruct((N, HD), jnp.float32),
            jax.ShapeDtypeStruct((_NCHP * _CN, DEG), jnp.float32),
        ],
    )(h, eig132, T, ws, wd)


def _agg_body(R_ref, w_ref, x_ref, xd_ref, sn_ref, preb_ref,
              p0_ref, p12_ref, p3_ref, p4_ref, pb_ref, postb_ref,
              h2_ref, stats_ref):
    i = pl.program_id(0)
    w = w_ref[...]
    x = x_ref[...]
    base = xd_ref[...] + preb_ref[...]
    gsum = R_ref[:, 0, :]
    gmax = R_ref[:, 1, :]
    gw = R_ref[:, 2, :]
    sw = jnp.sum(w, axis=1, keepdims=True)
    a_dir = jnp.abs(gw + sw * (base - x))
    h2 = (jnp.dot(x, p0_ref[...], preferred_element_type=jnp.float32)
          + jnp.dot(gsum, p12_ref[...], preferred_element_type=jnp.float32)
          + jnp.dot(gmax, p3_ref[...], preferred_element_type=jnp.float32)
          + jnp.dot(a_dir, p4_ref[...], preferred_element_type=jnp.float32)
          + jnp.dot(base, pb_ref[...], preferred_element_type=jnp.float32)
          + postb_ref[...]) * sn_ref[...]
    h2_ref[...] = h2

    @pl.when(i == 0)
    def _():
        stats_ref[...] = jnp.zeros_like(stats_ref)

    s1 = jnp.sum(h2, axis=0, keepdims=True)
    s2 = jnp.sum(h2 * h2, axis=0, keepdims=True)
    stats_ref[...] += jnp.concatenate([s1, s2], axis=0)


def _agg_call(R, w, x, xd, snorm, preb, p0, p12, p3, p4, pb, postb,
              nnodes, node_off):
    nb = nnodes // _NB
    off = node_off // _NB
    blk = lambda c: pl.BlockSpec((_NB, c), lambda i, o=off: (i + o, 0))
    full = lambda r, c: pl.BlockSpec((r, c), lambda i: (0, 0))
    return pl.pallas_call(
        _agg_body,
        grid=(nb,),
        in_specs=[
            pl.BlockSpec((_NB, 3, HD), lambda i: (i, 0, 0)),
            blk(DEG), blk(HD), blk(HD), blk(1),
            full(1, HD), full(HD, OD), full(HD, OD), full(HD, OD), full(HD, OD),
            full(HD, OD), full(1, OD),
        ],
        out_specs=[
            pl.BlockSpec((_NB, OD), lambda i: (i, 0)),
            pl.BlockSpec((2, OD), lambda i: (0, 0)),
        ],
        out_shape=[
            jax.ShapeDtypeStruct((nnodes, OD), jnp.float32),
            jax.ShapeDtypeStruct((2, OD), jnp.float32),
        ],
    )(R, w, x, xd, snorm, preb, p0, p12, p3, p4, pb, postb)


def _bn_scale(stats_ref, g_ref, b_ref):
    st = stats_ref[...]
    mu = st[0:1, :] * (1.0 / N)
    var = st[1:2, :] * (1.0 / N) - mu * mu
    scale = g_ref[...] * lax.rsqrt(var + 1e-5)
    return mu, scale, b_ref[...]


def _bnnext_body(h2_ref, stats_ref, g_ref, b_ref, ws_ref, wd_ref,
                 x_ref, xs_ref, xd_ref):
    mu, scale, b = _bn_scale(stats_ref, g_ref, b_ref)
    xn = jnp.maximum((h2_ref[...] - mu) * scale + b, 0.0)
    x_ref[...] = xn
    xs_ref[...] = jnp.dot(xn, ws_ref[...], preferred_element_type=jnp.float32)
    xd_ref[...] = jnp.dot(xn, wd_ref[...], preferred_element_type=jnp.float32)


def _bnnext_call(h2, stats, g, b, ws, wd):
    nb = N // _NBE
    full = lambda r, c: pl.BlockSpec((r, c), lambda i: (0, 0))
    return pl.pallas_call(
        _bnnext_body,
        grid=(nb,),
        in_specs=[
            pl.BlockSpec((_NBE, OD), lambda i: (i, 0)),
            full(2, OD), full(1, OD), full(1, OD), full(HD, HD), full(HD, HD),
        ],
        out_specs=[
            pl.BlockSpec((_NBE, HD), lambda i: (i, 0)),
            pl.BlockSpec((_NBE, HD), lambda i: (i, 0)),
            pl.BlockSpec((_NBE, HD), lambda i: (i, 0)),
        ],
        out_shape=[
            jax.ShapeDtypeStruct((N, HD), jnp.float32),
            jax.ShapeDtypeStruct((N, HD), jnp.float32),
            jax.ShapeDtypeStruct((N, HD), jnp.float32),
        ],
    )(h2, stats, g, b, ws, wd)


def _bnout_body(h2_ref, stats_ref, g_ref, b_ref,
                r0w_ref, r0b_ref, r1w_ref, r1b_ref, r2w_ref, r2b_ref,
                y_ref, hsum_ref):
    i = pl.program_id(0)
    mu, scale, b = _bn_scale(stats_ref, g_ref, b_ref)
    xn = jnp.maximum((h2_ref[...] - mu) * scale + b, 0.0)

    @pl.when(i == 0)
    def _():
        hsum_ref[...] = jnp.zeros_like(hsum_ref)

    hsum_ref[...] += jnp.sum(xn, axis=0, keepdims=True)

    @pl.when(i == (N // _NBE) - 1)
    def _():
        hg = hsum_ref[...] * (1.0 / N)
        y0 = jnp.maximum(
            jnp.dot(hg, r0w_ref[...], preferred_element_type=jnp.float32)
            + r0b_ref[...], 0.0)
        y1 = jnp.maximum(
            jnp.dot(y0, r1w_ref[...], preferred_element_type=jnp.float32)
            + r1b_ref[...], 0.0)
        y_ref[...] = (jnp.dot(y1, r2w_ref[...], preferred_element_type=jnp.float32)
                      + r2b_ref[...])


def _bnout_call(h2, stats, g, b, r0W, r0b, r1W, r1b, r2W, r2b):
    nb = N // _NBE
    full = lambda r, c: pl.BlockSpec((r, c), lambda i: (0, 0))
    return pl.pallas_call(
        _bnout_body,
        grid=(nb,),
        in_specs=[
            pl.BlockSpec((_NBE, OD), lambda i: (i, 0)),
            full(2, OD), full(1, OD), full(1, OD),
            full(OD, OD // 2), full(1, OD // 2),
            full(OD // 2, OD // 4), full(1, OD // 4),
            full(OD // 4, 128), full(1, 128),
        ],
        out_specs=pl.BlockSpec((1, 128), lambda i: (0, 0)),
        out_shape=jax.ShapeDtypeStruct((1, 128), jnp.float32),
        scratch_shapes=[pltpu.VMEM((1, OD), jnp.float32)],
    )(h2, stats, g, b, r0W, r0b, r1W, r1b, r2W, r2b)


def kernel(h, edge_index, eig, e, snorm_n, atom_emb, pre_W, pre_b, post_W,
           post_b, bn_g, bn_b, r0W, r0b, r1W, r1b, r2W, r2b):
    src = edge_index[0]
    src3 = jnp.pad(src, (0, _NCHP * _CE - E)).reshape(_NCHP, _CE)
    eig132 = eig.T[1].reshape(N, DEG)
    T = jnp.zeros((9, 128, HD), jnp.float32).at[:, :119, :].set(atom_emb)
    Ws = pre_W[:, :HD, :]
    Wd = pre_W[:, HD:, :]
    P0 = post_W[:, 0:HD]
    P12 = post_W[:, HD:2 * HD] / float(DEG) + post_W[:, 2 * HD:3 * HD]
    P3 = post_W[:, 3 * HD:4 * HD]
    P4 = post_W[:, 4 * HD:5 * HD]
    Pb = float(DEG) * P12 + P3
    preb = pre_b.reshape(NL, 1, HD)
    postb = post_b.reshape(NL, 1, OD)
    bng = bn_g.reshape(NL, 1, OD)
    bnb = bn_b.reshape(NL, 1, OD)

    h128 = jnp.pad(h, ((0, 0), (0, 128 - h.shape[1])))
    x, xs, xd, w = _enc_call(h128, eig132, T, Ws[0], Wd[0])
    gr = _sc_gr_kernel()
    y = None
    wflat = w.reshape(-1)
    for l in range(NL):
        R = gr(xs, src3, wflat)
        h2, stats = _agg_call(R, w, x, xd, snorm_n, preb[l],
                              P0[l], P12[l], P3[l], P4[l], Pb[l], postb[l],
                              N, 0)
        if l < NL - 1:
            x, xs, xd = _bnnext_call(h2, stats, bng[l], bnb[l],
                                     Ws[l + 1], Wd[l + 1])
        else:
            y = _bnout_call(h2, stats, bng[l], bnb[l],
                            r0W, r0b.reshape(1, -1), r1W, r1b.reshape(1, -1),
                            r2W, r2b.reshape(1, -1))
    return y


# speedup vs baseline: 1.1246x; 1.0455x over previous
import functools

import jax
import jax.numpy as jnp
from jax import lax
from jax.experimental import pallas as pl
from jax.experimental.pallas import tpu as pltpu
from jax.experimental.pallas import tpu_sc as plsc

N = 10000
DEG = 32
E = N * DEG
HD = 128
OD = 128
NL = 3

_NC, _NS = 2, 16
_NW = _NC * _NS
_CN = 4
_CE = _CN * DEG
_NCHUNK = N // _CN
_NCHP = 2504
_GRP = 8
_LANES = 16
_WCH = 80


def _sc_gr_body(cbase, npart, table_hbm, src_hbm, w_hbm, out_hbm, idx_v, w_v,
                rb0, rb1, stage, sem0, sem1):
    wid = lax.axis_index("s") * _NC + lax.axis_index("c")
    c0 = cbase + 8 * ((npart * wid + 8 * _NW - 1) // (8 * _NW))
    lc0 = c0 - cbase
    pltpu.sync_copy(src_hbm.at[pl.ds(c0, _WCH)], idx_v)
    pltpu.sync_copy(w_hbm.at[pl.ds(c0 * _CN * DEG, _WCH * _CN * DEG)], w_v)

    def start(lc, rb, sem):
        pltpu.async_copy(table_hbm.at[idx_v.at[lc]], rb, sem)

    def wait(lc, rb, sem):
        pltpu.make_async_copy(table_hbm.at[idx_v.at[lc]], rb, sem).wait()

    def compute(lc, srow0, rb):
        def node(j, carry):
            row0 = j * DEG
            wbase = (lc * _CN + j) * DEG
            wv = (w_v[pl.ds(wbase, _LANES)], w_v[pl.ds(wbase + _LANES, _LANES)])
            s = [None] * 8
            m = [None] * 8
            a = [None] * 8
            for k in range(DEG):
                wk = wv[k // _LANES][k % _LANES]
                for d in range(8):
                    v = rb[row0 + k, pl.ds(d * _LANES, _LANES)]
                    if k == 0:
                        s[d] = v
                        m[d] = v
                        a[d] = wk * v
                    else:
                        s[d] = s[d] + v
                        m[d] = jnp.maximum(m[d], v)
                        a[d] = a[d] + wk * v
            for d in range(8):
                sl = pl.ds(d * _LANES, _LANES)
                stage[srow0 + j, 0, sl] = s[d]
                stage[srow0 + j, 1, sl] = m[d]
                stage[srow0 + j, 2, sl] = a[d]
            return carry

        lax.fori_loop(0, _CN, node, 0)

    start(0, rb0, sem0)

    def group(g, carry):
        def pair(p, carry2):
            lcA = g * _GRP + 2 * p
            start(lcA + 1, rb1, sem1)
            wait(lcA, rb0, sem0)
            compute(lcA, 2 * p * _CN, rb0)

            @pl.when(lcA + 2 < _WCH)
            def _():
                start(lcA + 2, rb0, sem0)

            wait(lcA + 1, rb1, sem1)
            compute(lcA + 1, (2 * p + 1) * _CN, rb1)
            return carry2

        lax.fori_loop(0, _GRP // 2, pair, 0)
        pltpu.sync_copy(stage, out_hbm.at[pl.ds((lc0 + g * _GRP) * _CN,
                                                _GRP * _CN)])
        return carry

    lax.fori_loop(0, _WCH // _GRP, group, 0)


def _make_gr(cbase, npart, nout_chunks):
    scratch = [
        pltpu.VMEM((_WCH, _CE), jnp.int32),
        pltpu.VMEM((_WCH * _CN * DEG,), jnp.float32),
        pltpu.VMEM((_CE, HD), jnp.float32),
        pltpu.VMEM((_CE, HD), jnp.float32),
        pltpu.VMEM((_GRP * _CN, 3, HD), jnp.float32),
        pltpu.SemaphoreType.DMA,
        pltpu.SemaphoreType.DMA,
    ]
    return functools.partial(
        pl.kernel,
        out_type=jax.ShapeDtypeStruct((nout_chunks * _CN, 3, HD), jnp.float32),
        mesh=plsc.VectorSubcoreMesh(core_axis_name="c", subcore_axis_name="s"),
        scratch_types=scratch,
    )(functools.partial(_sc_gr_body, cbase, npart))


@functools.cache
def _sc_gr_kernel():
    return _make_gr(0, _NCHUNK, _NCHP)


_NBE = 1000
_NB = 400


def _enc_body(h_ref, eig_ref, T_ref, ws_ref, wd_ref,
              x_ref, xs_ref, xd_ref, w_ref):
    ids = lax.broadcasted_iota(jnp.int32, (_NBE, 128), 1)
    x = jnp.zeros((_NBE, HD), jnp.float32)
    for f in range(9):
        oh = (ids == h_ref[:, f][:, None]).astype(jnp.float32)
        x = x + jnp.dot(oh, T_ref[f], preferred_element_type=jnp.float32)
    x_ref[...] = x
    xs_ref[...] = jnp.dot(x, ws_ref[...], preferred_element_type=jnp.float32)
    xd_ref[...] = jnp.dot(x, wd_ref[...], preferred_element_type=jnp.float32)
    eig1 = eig_ref[...]
    absum = jnp.sum(jnp.abs(eig1), axis=1, keepdims=True) + 1e-8
    w_ref[...] = eig1 / absum


def _enc_call(h, eig132, T, ws, wd):
    nb = N // _NBE
    return pl.pallas_call(
        _enc_body,
        grid=(nb,),
        in_specs=[
            pl.BlockSpec((_NBE, 128), lambda i: (i, 0)),
            pl.BlockSpec((_NBE, DEG), lambda i: (i, 0)),
            pl.BlockSpec((9, 128, HD), lambda i: (0, 0, 0)),
            pl.BlockSpec((HD, HD), lambda i: (0, 0)),
            pl.BlockSpec((HD, HD), lambda i: (0, 0)),
        ],
        out_specs=[
            pl.BlockSpec((_NBE, HD), lambda i: (i, 0)),
            pl.BlockSpec((_NBE, HD), lambda i: (i, 0)),
            pl.BlockSpec((_NBE, HD), lambda i: (i, 0)),
            pl.BlockSpec((_NBE, DEG), lambda i: (i, 0)),
        ],
        out_shape=[
            jax.ShapeDtypeStruct((N, HD), jnp.float32),
            jax.ShapeDtypeStruct((N, HD), jnp.float32),
            jax.ShapeDtypeStruct((N, HD), jnp.float32),
            jax.ShapeDtypeStruct((_NCHP * _CN, DEG), jnp.float32),
        ],
    )(h, eig132, T, ws, wd)


def _agg_body(R_ref, w_ref, x_ref, xd_ref, sn_ref, preb_ref,
              p0_ref, p12_ref, p3_ref, p4_ref, pb_ref, postb_ref,
              h2_ref, stats_ref):
    i = pl.program_id(0)
    w = w_ref[...]
    x = x_ref[...]
    base = xd_ref[...] + preb_ref[...]
    gsum = R_ref[:, 0, :]
    gmax = R_ref[:, 1, :]
    gw = R_ref[:, 2, :]
    sw = jnp.sum(w, axis=1, keepdims=True)
    a_dir = jnp.abs(gw + sw * (base - x))
    h2 = (jnp.dot(x, p0_ref[...], preferred_element_type=jnp.float32)
          + jnp.dot(gsum, p12_ref[...], preferred_element_type=jnp.float32)
          + jnp.dot(gmax, p3_ref[...], preferred_element_type=jnp.float32)
          + jnp.dot(a_dir, p4_ref[...], preferred_element_type=jnp.float32)
          + jnp.dot(base, pb_ref[...], preferred_element_type=jnp.float32)
          + postb_ref[...]) * sn_ref[...]
    h2_ref[...] = h2

    @pl.when(i == 0)
    def _():
        stats_ref[...] = jnp.zeros_like(stats_ref)

    s1 = jnp.sum(h2, axis=0, keepdims=True)
    s2 = jnp.sum(h2 * h2, axis=0, keepdims=True)
    stats_ref[...] += jnp.concatenate([s1, s2], axis=0)


def _agg_call(R, w, x, xd, snorm, preb, p0, p12, p3, p4, pb, postb,
              nnodes, node_off):
    nb = nnodes // _NB
    off = node_off // _NB
    blk = lambda c: pl.BlockSpec((_NB, c), lambda i, o=off: (i + o, 0))
    full = lambda r, c: pl.BlockSpec((r, c), lambda i: (0, 0))
    return pl.pallas_call(
        _agg_body,
        grid=(nb,),
        in_specs=[
            pl.BlockSpec((_NB, 3, HD), lambda i: (i, 0, 0)),
            blk(DEG), blk(HD), blk(HD), blk(1),
            full(1, HD), full(HD, OD), full(HD, OD), full(HD, OD), full(HD, OD),
            full(HD, OD), full(1, OD),
        ],
        out_specs=[
            pl.BlockSpec((_NB, OD), lambda i: (i, 0)),
            pl.BlockSpec((2, OD), lambda i: (0, 0)),
        ],
        out_shape=[
            jax.ShapeDtypeStruct((nnodes, OD), jnp.float32),
            jax.ShapeDtypeStruct((2, OD), jnp.float32),
        ],
    )(R, w, x, xd, snorm, preb, p0, p12, p3, p4, pb, postb)


def _bn_scale(stats_ref, g_ref, b_ref):
    st = stats_ref[...]
    mu = st[0:1, :] * (1.0 / N)
    var = st[1:2, :] * (1.0 / N) - mu * mu
    scale = g_ref[...] * lax.rsqrt(var + 1e-5)
    return mu, scale, b_ref[...]


def _bnnext_body(h2_ref, stats_ref, g_ref, b_ref, ws_ref, wd_ref,
                 x_ref, xs_ref, xd_ref):
    mu, scale, b = _bn_scale(stats_ref, g_ref, b_ref)
    xn = jnp.maximum((h2_ref[...] - mu) * scale + b, 0.0)
    x_ref[...] = xn
    xs_ref[...] = jnp.dot(xn, ws_ref[...], preferred_element_type=jnp.float32)
    xd_ref[...] = jnp.dot(xn, wd_ref[...], preferred_element_type=jnp.float32)


def _bnnext_call(h2, stats, g, b, ws, wd):
    nb = N // _NBE
    full = lambda r, c: pl.BlockSpec((r, c), lambda i: (0, 0))
    return pl.pallas_call(
        _bnnext_body,
        grid=(nb,),
        in_specs=[
            pl.BlockSpec((_NBE, OD), lambda i: (i, 0)),
            full(2, OD), full(1, OD), full(1, OD), full(HD, HD), full(HD, HD),
        ],
        out_specs=[
            pl.BlockSpec((_NBE, HD), lambda i: (i, 0)),
            pl.BlockSpec((_NBE, HD), lambda i: (i, 0)),
            pl.BlockSpec((_NBE, HD), lambda i: (i, 0)),
        ],
        out_shape=[
            jax.ShapeDtypeStruct((N, HD), jnp.float32),
            jax.ShapeDtypeStruct((N, HD), jnp.float32),
            jax.ShapeDtypeStruct((N, HD), jnp.float32),
        ],
    )(h2, stats, g, b, ws, wd)


def _bnout_body(h2_ref, stats_ref, g_ref, b_ref,
                r0w_ref, r0b_ref, r1w_ref, r1b_ref, r2w_ref, r2b_ref,
                y_ref, hsum_ref):
    i = pl.program_id(0)
    mu, scale, b = _bn_scale(stats_ref, g_ref, b_ref)
    xn = jnp.maximum((h2_ref[...] - mu) * scale + b, 0.0)

    @pl.when(i == 0)
    def _():
        hsum_ref[...] = jnp.zeros_like(hsum_ref)

    hsum_ref[...] += jnp.sum(xn, axis=0, keepdims=True)

    @pl.when(i == (N // _NBE) - 1)
    def _():
        hg = hsum_ref[...] * (1.0 / N)
        y0 = jnp.maximum(
            jnp.dot(hg, r0w_ref[...], preferred_element_type=jnp.float32)
            + r0b_ref[...], 0.0)
        y1 = jnp.maximum(
            jnp.dot(y0, r1w_ref[...], preferred_element_type=jnp.float32)
            + r1b_ref[...], 0.0)
        y_ref[...] = (jnp.dot(y1, r2w_ref[...], preferred_element_type=jnp.float32)
                      + r2b_ref[...])


def _bnout_call(h2, stats, g, b, r0W, r0b, r1W, r1b, r2W, r2b):
    nb = N // _NBE
    full = lambda r, c: pl.BlockSpec((r, c), lambda i: (0, 0))
    return pl.pallas_call(
        _bnout_body,
        grid=(nb,),
        in_specs=[
            pl.BlockSpec((_NBE, OD), lambda i: (i, 0)),
            full(2, OD), full(1, OD), full(1, OD),
            full(OD, OD // 2), full(1, OD // 2),
            full(OD // 2, OD // 4), full(1, OD // 4),
            full(OD // 4, 128), full(1, 128),
        ],
        out_specs=pl.BlockSpec((1, 128), lambda i: (0, 0)),
        out_shape=jax.ShapeDtypeStruct((1, 128), jnp.float32),
        scratch_shapes=[pltpu.VMEM((1, OD), jnp.float32)],
    )(h2, stats, g, b, r0W, r0b, r1W, r1b, r2W, r2b)


def kernel(h, edge_index, eig, e, snorm_n, atom_emb, pre_W, pre_b, post_W,
           post_b, bn_g, bn_b, r0W, r0b, r1W, r1b, r2W, r2b):
    src = edge_index[0]
    src3 = jnp.pad(src, (0, _NCHP * _CE - E)).reshape(_NCHP, _CE)
    eig132 = eig.T[1].reshape(N, DEG)
    T = jnp.zeros((9, 128, HD), jnp.float32).at[:, :119, :].set(atom_emb)
    Ws = pre_W[:, :HD, :]
    Wd = pre_W[:, HD:, :]
    P0 = post_W[:, 0:HD]
    P12 = post_W[:, HD:2 * HD] / float(DEG) + post_W[:, 2 * HD:3 * HD]
    P3 = post_W[:, 3 * HD:4 * HD]
    P4 = post_W[:, 4 * HD:5 * HD]
    Pb = float(DEG) * P12 + P3
    preb = pre_b.reshape(NL, 1, HD)
    postb = post_b.reshape(NL, 1, OD)
    bng = bn_g.reshape(NL, 1, OD)
    bnb = bn_b.reshape(NL, 1, OD)

    h128 = jnp.pad(h, ((0, 0), (0, 128 - h.shape[1])))
    x, xs, xd, w = _enc_call(h128, eig132, T, Ws[0], Wd[0])
    gr = _sc_gr_kernel()
    y = None
    wflat = w.reshape(-1)
    for l in range(NL):
        R = gr(xs, src3, wflat)
        h2, stats = _agg_call(R, w, x, xd, snorm_n, preb[l],
                              P0[l], P12[l], P3[l], P4[l], Pb[l], postb[l],
                              N, 0)
        if l < NL - 1:
            x, xs, xd = _bnnext_call(h2, stats, bng[l], bnb[l],
                                     Ws[l + 1], Wd[l + 1])
        else:
            y = _bnout_call(h2, stats, bng[l], bnb[l],
                            r0W, r0b.reshape(1, -1), r1W, r1b.reshape(1, -1),
                            r2W, r2b.reshape(1, -1))
    return y


# speedup vs baseline: 1.1248x; 1.0002x over previous
import functools

import jax
import jax.numpy as jnp
from jax import lax
from jax.experimental import pallas as pl
from jax.experimental.pallas import tpu as pltpu
from jax.experimental.pallas import tpu_sc as plsc

N = 10000
DEG = 32
E = N * DEG
HD = 128
OD = 128
NL = 3

_NC, _NS = 2, 16
_NW = _NC * _NS
_CN = 4
_CE = _CN * DEG
_NCHUNK = N // _CN
_NCHP = 2504
_GRP = 16
_LANES = 16
_WCH = 80


def _sc_gr_body(cbase, npart, table_hbm, src_hbm, w_hbm, out_hbm, idx_v, w_v,
                rb0, rb1, stage, sem0, sem1):
    wid = lax.axis_index("s") * _NC + lax.axis_index("c")
    c0 = cbase + 8 * ((npart * wid + 8 * _NW - 1) // (8 * _NW))
    lc0 = c0 - cbase
    pltpu.sync_copy(src_hbm.at[pl.ds(c0, _WCH)], idx_v)
    pltpu.sync_copy(w_hbm.at[pl.ds(c0 * _CN * DEG, _WCH * _CN * DEG)], w_v)

    def start(lc, rb, sem):
        pltpu.async_copy(table_hbm.at[idx_v.at[lc]], rb, sem)

    def wait(lc, rb, sem):
        pltpu.make_async_copy(table_hbm.at[idx_v.at[lc]], rb, sem).wait()

    def compute(lc, srow0, rb):
        def node(j, carry):
            row0 = j * DEG
            wbase = (lc * _CN + j) * DEG
            wv = (w_v[pl.ds(wbase, _LANES)], w_v[pl.ds(wbase + _LANES, _LANES)])
            s = [None] * 8
            m = [None] * 8
            a = [None] * 8
            for k in range(DEG):
                wk = wv[k // _LANES][k % _LANES]
                for d in range(8):
                    v = rb[row0 + k, pl.ds(d * _LANES, _LANES)]
                    if k == 0:
                        s[d] = v
                        m[d] = v
                        a[d] = wk * v
                    else:
                        s[d] = s[d] + v
                        m[d] = jnp.maximum(m[d], v)
                        a[d] = a[d] + wk * v
            for d in range(8):
                sl = pl.ds(d * _LANES, _LANES)
                stage[srow0 + j, 0, sl] = s[d]
                stage[srow0 + j, 1, sl] = m[d]
                stage[srow0 + j, 2, sl] = a[d]
            return carry

        lax.fori_loop(0, _CN, node, 0)

    start(0, rb0, sem0)

    def group(g, carry):
        def pair(p, carry2):
            lcA = g * _GRP + 2 * p
            start(lcA + 1, rb1, sem1)
            wait(lcA, rb0, sem0)
            compute(lcA, 2 * p * _CN, rb0)

            @pl.when(lcA + 2 < _WCH)
            def _():
                start(lcA + 2, rb0, sem0)

            wait(lcA + 1, rb1, sem1)
            compute(lcA + 1, (2 * p + 1) * _CN, rb1)
            return carry2

        lax.fori_loop(0, _GRP // 2, pair, 0)
        pltpu.sync_copy(stage, out_hbm.at[pl.ds((lc0 + g * _GRP) * _CN,
                                                _GRP * _CN)])
        return carry

    lax.fori_loop(0, _WCH // _GRP, group, 0)


def _make_gr(cbase, npart, nout_chunks):
    scratch = [
        pltpu.VMEM((_WCH, _CE), jnp.int32),
        pltpu.VMEM((_WCH * _CN * DEG,), jnp.float32),
        pltpu.VMEM((_CE, HD), jnp.float32),
        pltpu.VMEM((_CE, HD), jnp.float32),
        pltpu.VMEM((_GRP * _CN, 3, HD), jnp.float32),
        pltpu.SemaphoreType.DMA,
        pltpu.SemaphoreType.DMA,
    ]
    return functools.partial(
        pl.kernel,
        out_type=jax.ShapeDtypeStruct((nout_chunks * _CN, 3, HD), jnp.float32),
        mesh=plsc.VectorSubcoreMesh(core_axis_name="c", subcore_axis_name="s"),
        scratch_types=scratch,
    )(functools.partial(_sc_gr_body, cbase, npart))


@functools.cache
def _sc_gr_kernel():
    return _make_gr(0, _NCHUNK, _NCHP)


_NBE = 1000
_NB = 400


def _enc_body(h_ref, eig_ref, T_ref, ws_ref, wd_ref,
              x_ref, xs_ref, xd_ref, w_ref):
    ids = lax.broadcasted_iota(jnp.int32, (_NBE, 128), 1)
    x = jnp.zeros((_NBE, HD), jnp.float32)
    for f in range(9):
        oh = (ids == h_ref[:, f][:, None]).astype(jnp.float32)
        x = x + jnp.dot(oh, T_ref[f], preferred_element_type=jnp.float32)
    x_ref[...] = x
    xs_ref[...] = jnp.dot(x, ws_ref[...], preferred_element_type=jnp.float32)
    xd_ref[...] = jnp.dot(x, wd_ref[...], preferred_element_type=jnp.float32)
    eig1 = eig_ref[...]
    absum = jnp.sum(jnp.abs(eig1), axis=1, keepdims=True) + 1e-8
    w_ref[...] = eig1 / absum


def _enc_call(h, eig132, T, ws, wd):
    nb = N // _NBE
    return pl.pallas_call(
        _enc_body,
        grid=(nb,),
        in_specs=[
            pl.BlockSpec((_NBE, 128), lambda i: (i, 0)),
            pl.BlockSpec((_NBE, DEG), lambda i: (i, 0)),
            pl.BlockSpec((9, 128, HD), lambda i: (0, 0, 0)),
            pl.BlockSpec((HD, HD), lambda i: (0, 0)),
            pl.BlockSpec((HD, HD), lambda i: (0, 0)),
        ],
        out_specs=[
            pl.BlockSpec((_NBE, HD), lambda i: (i, 0)),
            pl.BlockSpec((_NBE, HD), lambda i: (i, 0)),
            pl.BlockSpec((_NBE, HD), lambda i: (i, 0)),
            pl.BlockSpec((_NBE, DEG), lambda i: (i, 0)),
        ],
        out_shape=[
            jax.ShapeDtypeStruct((N, HD), jnp.float32),
            jax.ShapeDtypeStruct((N, HD), jnp.float32),
            jax.ShapeDtypeStruct((N, HD), jnp.float32),
            jax.ShapeDtypeStruct((_NCHP * _CN, DEG), jnp.float32),
        ],
    )(h, eig132, T, ws, wd)


def _agg_body(R_ref, w_ref, x_ref, xd_ref, sn_ref, preb_ref,
              p0_ref, p12_ref, p3_ref, p4_ref, pb_ref, postb_ref,
              h2_ref, stats_ref):
    i = pl.program_id(0)
    w = w_ref[...]
    x = x_ref[...]
    base = xd_ref[...] + preb_ref[...]
    gsum = R_ref[:, 0, :]
    gmax = R_ref[:, 1, :]
    gw = R_ref[:, 2, :]
    sw = jnp.sum(w, axis=1, keepdims=True)
    a_dir = jnp.abs(gw + sw * (base - x))
    h2 = (jnp.dot(x, p0_ref[...], preferred_element_type=jnp.float32)
          + jnp.dot(gsum, p12_ref[...], preferred_element_type=jnp.float32)
          + jnp.dot(gmax, p3_ref[...], preferred_element_type=jnp.float32)
          + jnp.dot(a_dir, p4_ref[...], preferred_element_type=jnp.float32)
          + jnp.dot(base, pb_ref[...], preferred_element_type=jnp.float32)
          + postb_ref[...]) * sn_ref[...]
    h2_ref[...] = h2

    @pl.when(i == 0)
    def _():
        stats_ref[...] = jnp.zeros_like(stats_ref)

    s1 = jnp.sum(h2, axis=0, keepdims=True)
    s2 = jnp.sum(h2 * h2, axis=0, keepdims=True)
    stats_ref[...] += jnp.concatenate([s1, s2], axis=0)


def _agg_call(R, w, x, xd, snorm, preb, p0, p12, p3, p4, pb, postb,
              nnodes, node_off):
    nb = nnodes // _NB
    off = node_off // _NB
    blk = lambda c: pl.BlockSpec((_NB, c), lambda i, o=off: (i + o, 0))
    full = lambda r, c: pl.BlockSpec((r, c), lambda i: (0, 0))
    return pl.pallas_call(
        _agg_body,
        grid=(nb,),
        in_specs=[
            pl.BlockSpec((_NB, 3, HD), lambda i: (i, 0, 0)),
            blk(DEG), blk(HD), blk(HD), blk(1),
            full(1, HD), full(HD, OD), full(HD, OD), full(HD, OD), full(HD, OD),
            full(HD, OD), full(1, OD),
        ],
        out_specs=[
            pl.BlockSpec((_NB, OD), lambda i: (i, 0)),
            pl.BlockSpec((2, OD), lambda i: (0, 0)),
        ],
        out_shape=[
            jax.ShapeDtypeStruct((nnodes, OD), jnp.float32),
            jax.ShapeDtypeStruct((2, OD), jnp.float32),
        ],
    )(R, w, x, xd, snorm, preb, p0, p12, p3, p4, pb, postb)


def _bn_scale(stats_ref, g_ref, b_ref):
    st = stats_ref[...]
    mu = st[0:1, :] * (1.0 / N)
    var = st[1:2, :] * (1.0 / N) - mu * mu
    scale = g_ref[...] * lax.rsqrt(var + 1e-5)
    return mu, scale, b_ref[...]


def _bnnext_body(h2_ref, stats_ref, g_ref, b_ref, ws_ref, wd_ref,
                 x_ref, xs_ref, xd_ref):
    mu, scale, b = _bn_scale(stats_ref, g_ref, b_ref)
    xn = jnp.maximum((h2_ref[...] - mu) * scale + b, 0.0)
    x_ref[...] = xn
    xs_ref[...] = jnp.dot(xn, ws_ref[...], preferred_element_type=jnp.float32)
    xd_ref[...] = jnp.dot(xn, wd_ref[...], preferred_element_type=jnp.float32)


def _bnnext_call(h2, stats, g, b, ws, wd):
    nb = N // _NBE
    full = lambda r, c: pl.BlockSpec((r, c), lambda i: (0, 0))
    return pl.pallas_call(
        _bnnext_body,
        grid=(nb,),
        in_specs=[
            pl.BlockSpec((_NBE, OD), lambda i: (i, 0)),
            full(2, OD), full(1, OD), full(1, OD), full(HD, HD), full(HD, HD),
        ],
        out_specs=[
            pl.BlockSpec((_NBE, HD), lambda i: (i, 0)),
            pl.BlockSpec((_NBE, HD), lambda i: (i, 0)),
            pl.BlockSpec((_NBE, HD), lambda i: (i, 0)),
        ],
        out_shape=[
            jax.ShapeDtypeStruct((N, HD), jnp.float32),
            jax.ShapeDtypeStruct((N, HD), jnp.float32),
            jax.ShapeDtypeStruct((N, HD), jnp.float32),
        ],
    )(h2, stats, g, b, ws, wd)


def _bnout_body(h2_ref, stats_ref, g_ref, b_ref,
                r0w_ref, r0b_ref, r1w_ref, r1b_ref, r2w_ref, r2b_ref,
                y_ref, hsum_ref):
    i = pl.program_id(0)
    mu, scale, b = _bn_scale(stats_ref, g_ref, b_ref)
    xn = jnp.maximum((h2_ref[...] - mu) * scale + b, 0.0)

    @pl.when(i == 0)
    def _():
        hsum_ref[...] = jnp.zeros_like(hsum_ref)

    hsum_ref[...] += jnp.sum(xn, axis=0, keepdims=True)

    @pl.when(i == (N // _NBE) - 1)
    def _():
        hg = hsum_ref[...] * (1.0 / N)
        y0 = jnp.maximum(
            jnp.dot(hg, r0w_ref[...], preferred_element_type=jnp.float32)
            + r0b_ref[...], 0.0)
        y1 = jnp.maximum(
            jnp.dot(y0, r1w_ref[...], preferred_element_type=jnp.float32)
            + r1b_ref[...], 0.0)
        y_ref[...] = (jnp.dot(y1, r2w_ref[...], preferred_element_type=jnp.float32)
                      + r2b_ref[...])


def _bnout_call(h2, stats, g, b, r0W, r0b, r1W, r1b, r2W, r2b):
    nb = N // _NBE
    full = lambda r, c: pl.BlockSpec((r, c), lambda i: (0, 0))
    return pl.pallas_call(
        _bnout_body,
        grid=(nb,),
        in_specs=[
            pl.BlockSpec((_NBE, OD), lambda i: (i, 0)),
            full(2, OD), full(1, OD), full(1, OD),
            full(OD, OD // 2), full(1, OD // 2),
            full(OD // 2, OD // 4), full(1, OD // 4),
            full(OD // 4, 128), full(1, 128),
        ],
        out_specs=pl.BlockSpec((1, 128), lambda i: (0, 0)),
        out_shape=jax.ShapeDtypeStruct((1, 128), jnp.float32),
        scratch_shapes=[pltpu.VMEM((1, OD), jnp.float32)],
    )(h2, stats, g, b, r0W, r0b, r1W, r1b, r2W, r2b)


def kernel(h, edge_index, eig, e, snorm_n, atom_emb, pre_W, pre_b, post_W,
           post_b, bn_g, bn_b, r0W, r0b, r1W, r1b, r2W, r2b):
    src = edge_index[0]
    src3 = jnp.pad(src, (0, _NCHP * _CE - E)).reshape(_NCHP, _CE)
    eig132 = eig.T[1].reshape(N, DEG)
    T = jnp.zeros((9, 128, HD), jnp.float32).at[:, :119, :].set(atom_emb)
    Ws = pre_W[:, :HD, :]
    Wd = pre_W[:, HD:, :]
    P0 = post_W[:, 0:HD]
    P12 = post_W[:, HD:2 * HD] / float(DEG) + post_W[:, 2 * HD:3 * HD]
    P3 = post_W[:, 3 * HD:4 * HD]
    P4 = post_W[:, 4 * HD:5 * HD]
    Pb = float(DEG) * P12 + P3
    preb = pre_b.reshape(NL, 1, HD)
    postb = post_b.reshape(NL, 1, OD)
    bng = bn_g.reshape(NL, 1, OD)
    bnb = bn_b.reshape(NL, 1, OD)

    h128 = jnp.pad(h, ((0, 0), (0, 128 - h.shape[1])))
    x, xs, xd, w = _enc_call(h128, eig132, T, Ws[0], Wd[0])
    gr = _sc_gr_kernel()
    y = None
    wflat = w.reshape(-1)
    for l in range(NL):
        R = gr(xs, src3, wflat)
        h2, stats = _agg_call(R, w, x, xd, snorm_n, preb[l],
                              P0[l], P12[l], P3[l], P4[l], Pb[l], postb[l],
                              N, 0)
        if l < NL - 1:
            x, xs, xd = _bnnext_call(h2, stats, bng[l], bnb[l],
                                     Ws[l + 1], Wd[l + 1])
        else:
            y = _bnout_call(h2, stats, bng[l], bnb[l],
                            r0W, r0b.reshape(1, -1), r1W, r1b.reshape(1, -1),
                            r2W, r2b.reshape(1, -1))
    return y


# speedup vs baseline: 1.1258x; 1.0009x over previous
import functools

import jax
import jax.numpy as jnp
from jax import lax
from jax.experimental import pallas as pl
from jax.experimental.pallas import tpu as pltpu
from jax.experimental.pallas import tpu_sc as plsc

N = 10000
DEG = 32
E = N * DEG
HD = 128
OD = 128
NL = 3

_NC, _NS = 2, 16
_NW = _NC * _NS
_CN = 4
_CE = _CN * DEG
_NCHUNK = N // _CN
_NCHP = 2504
_GRP = 16
_LANES = 16
_WCH = 80


def _sc_gr_body(cbase, npart, table_hbm, src_hbm, w_hbm, out_hbm, idx_v, w_v,
                rb0, rb1, stage, sem0, sem1):
    wid = lax.axis_index("s") * _NC + lax.axis_index("c")
    c0 = cbase + 8 * ((npart * wid + 8 * _NW - 1) // (8 * _NW))
    lc0 = c0 - cbase
    pltpu.sync_copy(src_hbm.at[pl.ds(c0, _WCH)], idx_v)
    pltpu.sync_copy(w_hbm.at[pl.ds(c0 * _CN * DEG, _WCH * _CN * DEG)], w_v)

    def start(lc, rb, sem):
        pltpu.async_copy(table_hbm.at[idx_v.at[lc]], rb, sem)

    def wait(lc, rb, sem):
        pltpu.make_async_copy(table_hbm.at[idx_v.at[lc]], rb, sem).wait()

    def compute(lc, srow0, rb):
        def node(j, carry):
            row0 = j * DEG
            wbase = (lc * _CN + j) * DEG
            wv = (w_v[pl.ds(wbase, _LANES)], w_v[pl.ds(wbase + _LANES, _LANES)])
            s = [[None] * 8 for _ in range(2)]
            m = [[None] * 8 for _ in range(2)]
            a = [[None] * 8 for _ in range(2)]
            for k in range(DEG):
                h_ = k // _LANES
                wk = wv[h_][k % _LANES]
                for d in range(8):
                    v = rb[row0 + k, pl.ds(d * _LANES, _LANES)]
                    if k % _LANES == 0:
                        s[h_][d] = v
                        m[h_][d] = v
                        a[h_][d] = wk * v
                    else:
                        s[h_][d] = s[h_][d] + v
                        m[h_][d] = jnp.maximum(m[h_][d], v)
                        a[h_][d] = a[h_][d] + wk * v
            for d in range(8):
                sl = pl.ds(d * _LANES, _LANES)
                stage[srow0 + j, 0, sl] = s[0][d] + s[1][d]
                stage[srow0 + j, 1, sl] = jnp.maximum(m[0][d], m[1][d])
                stage[srow0 + j, 2, sl] = a[0][d] + a[1][d]
            return carry

        lax.fori_loop(0, _CN, node, 0)

    start(0, rb0, sem0)

    def group(g, carry):
        def pair(p, carry2):
            lcA = g * _GRP + 2 * p
            start(lcA + 1, rb1, sem1)
            wait(lcA, rb0, sem0)
            compute(lcA, 2 * p * _CN, rb0)

            @pl.when(lcA + 2 < _WCH)
            def _():
                start(lcA + 2, rb0, sem0)

            wait(lcA + 1, rb1, sem1)
            compute(lcA + 1, (2 * p + 1) * _CN, rb1)
            return carry2

        lax.fori_loop(0, _GRP // 2, pair, 0)
        pltpu.sync_copy(stage, out_hbm.at[pl.ds((lc0 + g * _GRP) * _CN,
                                                _GRP * _CN)])
        return carry

    lax.fori_loop(0, _WCH // _GRP, group, 0)


def _make_gr(cbase, npart, nout_chunks):
    scratch = [
        pltpu.VMEM((_WCH, _CE), jnp.int32),
        pltpu.VMEM((_WCH * _CN * DEG,), jnp.float32),
        pltpu.VMEM((_CE, HD), jnp.float32),
        pltpu.VMEM((_CE, HD), jnp.float32),
        pltpu.VMEM((_GRP * _CN, 3, HD), jnp.float32),
        pltpu.SemaphoreType.DMA,
        pltpu.SemaphoreType.DMA,
    ]
    return functools.partial(
        pl.kernel,
        out_type=jax.ShapeDtypeStruct((nout_chunks * _CN, 3, HD), jnp.float32),
        mesh=plsc.VectorSubcoreMesh(core_axis_name="c", subcore_axis_name="s"),
        scratch_types=scratch,
    )(functools.partial(_sc_gr_body, cbase, npart))


@functools.cache
def _sc_gr_kernel():
    return _make_gr(0, _NCHUNK, _NCHP)


_NBE = 1000
_NB = 400


def _enc_body(h_ref, eig_ref, T_ref, ws_ref, wd_ref,
              x_ref, xs_ref, xd_ref, w_ref):
    ids = lax.broadcasted_iota(jnp.int32, (_NBE, 128), 1)
    x = jnp.zeros((_NBE, HD), jnp.float32)
    for f in range(9):
        oh = (ids == h_ref[:, f][:, None]).astype(jnp.float32)
        x = x + jnp.dot(oh, T_ref[f], preferred_element_type=jnp.float32)
    x_ref[...] = x
    xs_ref[...] = jnp.dot(x, ws_ref[...], preferred_element_type=jnp.float32)
    xd_ref[...] = jnp.dot(x, wd_ref[...], preferred_element_type=jnp.float32)
    eig1 = eig_ref[...]
    absum = jnp.sum(jnp.abs(eig1), axis=1, keepdims=True) + 1e-8
    w_ref[...] = eig1 / absum


def _enc_call(h, eig132, T, ws, wd):
    nb = N // _NBE
    return pl.pallas_call(
        _enc_body,
        grid=(nb,),
        in_specs=[
            pl.BlockSpec((_NBE, 128), lambda i: (i, 0)),
            pl.BlockSpec((_NBE, DEG), lambda i: (i, 0)),
            pl.BlockSpec((9, 128, HD), lambda i: (0, 0, 0)),
            pl.BlockSpec((HD, HD), lambda i: (0, 0)),
            pl.BlockSpec((HD, HD), lambda i: (0, 0)),
        ],
        out_specs=[
            pl.BlockSpec((_NBE, HD), lambda i: (i, 0)),
            pl.BlockSpec((_NBE, HD), lambda i: (i, 0)),
            pl.BlockSpec((_NBE, HD), lambda i: (i, 0)),
            pl.BlockSpec((_NBE, DEG), lambda i: (i, 0)),
        ],
        out_shape=[
            jax.ShapeDtypeStruct((N, HD), jnp.float32),
            jax.ShapeDtypeStruct((N, HD), jnp.float32),
            jax.ShapeDtypeStruct((N, HD), jnp.float32),
            jax.ShapeDtypeStruct((_NCHP * _CN, DEG), jnp.float32),
        ],
    )(h, eig132, T, ws, wd)


def _agg_body(R_ref, w_ref, x_ref, xd_ref, sn_ref, preb_ref,
              p0_ref, p12_ref, p3_ref, p4_ref, pb_ref, postb_ref,
              h2_ref, stats_ref):
    i = pl.program_id(0)
    w = w_ref[...]
    x = x_ref[...]
    base = xd_ref[...] + preb_ref[...]
    gsum = R_ref[:, 0, :]
    gmax = R_ref[:, 1, :]
    gw = R_ref[:, 2, :]
    sw = jnp.sum(w, axis=1, keepdims=True)
    a_dir = jnp.abs(gw + sw * (base - x))
    h2 = (jnp.dot(x, p0_ref[...], preferred_element_type=jnp.float32)
          + jnp.dot(gsum, p12_ref[...], preferred_element_type=jnp.float32)
          + jnp.dot(gmax, p3_ref[...], preferred_element_type=jnp.float32)
          + jnp.dot(a_dir, p4_ref[...], preferred_element_type=jnp.float32)
          + jnp.dot(base, pb_ref[...], preferred_element_type=jnp.float32)
          + postb_ref[...]) * sn_ref[...]
    h2_ref[...] = h2

    @pl.when(i == 0)
    def _():
        stats_ref[...] = jnp.zeros_like(stats_ref)

    s1 = jnp.sum(h2, axis=0, keepdims=True)
    s2 = jnp.sum(h2 * h2, axis=0, keepdims=True)
    stats_ref[...] += jnp.concatenate([s1, s2], axis=0)


def _agg_call(R, w, x, xd, snorm, preb, p0, p12, p3, p4, pb, postb,
              nnodes, node_off):
    nb = nnodes // _NB
    off = node_off // _NB
    blk = lambda c: pl.BlockSpec((_NB, c), lambda i, o=off: (i + o, 0))
    full = lambda r, c: pl.BlockSpec((r, c), lambda i: (0, 0))
    return pl.pallas_call(
        _agg_body,
        grid=(nb,),
        in_specs=[
            pl.BlockSpec((_NB, 3, HD), lambda i: (i, 0, 0)),
            blk(DEG), blk(HD), blk(HD), blk(1),
            full(1, HD), full(HD, OD), full(HD, OD), full(HD, OD), full(HD, OD),
            full(HD, OD), full(1, OD),
        ],
        out_specs=[
            pl.BlockSpec((_NB, OD), lambda i: (i, 0)),
            pl.BlockSpec((2, OD), lambda i: (0, 0)),
        ],
        out_shape=[
            jax.ShapeDtypeStruct((nnodes, OD), jnp.float32),
            jax.ShapeDtypeStruct((2, OD), jnp.float32),
        ],
    )(R, w, x, xd, snorm, preb, p0, p12, p3, p4, pb, postb)


def _bn_scale(stats_ref, g_ref, b_ref):
    st = stats_ref[...]
    mu = st[0:1, :] * (1.0 / N)
    var = st[1:2, :] * (1.0 / N) - mu * mu
    scale = g_ref[...] * lax.rsqrt(var + 1e-5)
    return mu, scale, b_ref[...]


def _bnnext_body(h2_ref, stats_ref, g_ref, b_ref, ws_ref, wd_ref,
                 x_ref, xs_ref, xd_ref):
    mu, scale, b = _bn_scale(stats_ref, g_ref, b_ref)
    xn = jnp.maximum((h2_ref[...] - mu) * scale + b, 0.0)
    x_ref[...] = xn
    xs_ref[...] = jnp.dot(xn, ws_ref[...], preferred_element_type=jnp.float32)
    xd_ref[...] = jnp.dot(xn, wd_ref[...], preferred_element_type=jnp.float32)


def _bnnext_call(h2, stats, g, b, ws, wd):
    nb = N // _NBE
    full = lambda r, c: pl.BlockSpec((r, c), lambda i: (0, 0))
    return pl.pallas_call(
        _bnnext_body,
        grid=(nb,),
        in_specs=[
            pl.BlockSpec((_NBE, OD), lambda i: (i, 0)),
            full(2, OD), full(1, OD), full(1, OD), full(HD, HD), full(HD, HD),
        ],
        out_specs=[
            pl.BlockSpec((_NBE, HD), lambda i: (i, 0)),
            pl.BlockSpec((_NBE, HD), lambda i: (i, 0)),
            pl.BlockSpec((_NBE, HD), lambda i: (i, 0)),
        ],
        out_shape=[
            jax.ShapeDtypeStruct((N, HD), jnp.float32),
            jax.ShapeDtypeStruct((N, HD), jnp.float32),
            jax.ShapeDtypeStruct((N, HD), jnp.float32),
        ],
    )(h2, stats, g, b, ws, wd)


def _bnout_body(h2_ref, stats_ref, g_ref, b_ref,
                r0w_ref, r0b_ref, r1w_ref, r1b_ref, r2w_ref, r2b_ref,
                y_ref, hsum_ref):
    i = pl.program_id(0)
    mu, scale, b = _bn_scale(stats_ref, g_ref, b_ref)
    xn = jnp.maximum((h2_ref[...] - mu) * scale + b, 0.0)

    @pl.when(i == 0)
    def _():
        hsum_ref[...] = jnp.zeros_like(hsum_ref)

    hsum_ref[...] += jnp.sum(xn, axis=0, keepdims=True)

    @pl.when(i == (N // _NBE) - 1)
    def _():
        hg = hsum_ref[...] * (1.0 / N)
        y0 = jnp.maximum(
            jnp.dot(hg, r0w_ref[...], preferred_element_type=jnp.float32)
            + r0b_ref[...], 0.0)
        y1 = jnp.maximum(
            jnp.dot(y0, r1w_ref[...], preferred_element_type=jnp.float32)
            + r1b_ref[...], 0.0)
        y_ref[...] = (jnp.dot(y1, r2w_ref[...], preferred_element_type=jnp.float32)
                      + r2b_ref[...])


def _bnout_call(h2, stats, g, b, r0W, r0b, r1W, r1b, r2W, r2b):
    nb = N // _NBE
    full = lambda r, c: pl.BlockSpec((r, c), lambda i: (0, 0))
    return pl.pallas_call(
        _bnout_body,
        grid=(nb,),
        in_specs=[
            pl.BlockSpec((_NBE, OD), lambda i: (i, 0)),
            full(2, OD), full(1, OD), full(1, OD),
            full(OD, OD // 2), full(1, OD // 2),
            full(OD // 2, OD // 4), full(1, OD // 4),
            full(OD // 4, 128), full(1, 128),
        ],
        out_specs=pl.BlockSpec((1, 128), lambda i: (0, 0)),
        out_shape=jax.ShapeDtypeStruct((1, 128), jnp.float32),
        scratch_shapes=[pltpu.VMEM((1, OD), jnp.float32)],
    )(h2, stats, g, b, r0W, r0b, r1W, r1b, r2W, r2b)


def kernel(h, edge_index, eig, e, snorm_n, atom_emb, pre_W, pre_b, post_W,
           post_b, bn_g, bn_b, r0W, r0b, r1W, r1b, r2W, r2b):
    src = edge_index[0]
    src3 = jnp.pad(src, (0, _NCHP * _CE - E)).reshape(_NCHP, _CE)
    eig132 = eig.T[1].reshape(N, DEG)
    T = jnp.zeros((9, 128, HD), jnp.float32).at[:, :119, :].set(atom_emb)
    Ws = pre_W[:, :HD, :]
    Wd = pre_W[:, HD:, :]
    P0 = post_W[:, 0:HD]
    P12 = post_W[:, HD:2 * HD] / float(DEG) + post_W[:, 2 * HD:3 * HD]
    P3 = post_W[:, 3 * HD:4 * HD]
    P4 = post_W[:, 4 * HD:5 * HD]
    Pb = float(DEG) * P12 + P3
    preb = pre_b.reshape(NL, 1, HD)
    postb = post_b.reshape(NL, 1, OD)
    bng = bn_g.reshape(NL, 1, OD)
    bnb = bn_b.reshape(NL, 1, OD)

    h128 = jnp.pad(h, ((0, 0), (0, 128 - h.shape[1])))
    x, xs, xd, w = _enc_call(h128, eig132, T, Ws[0], Wd[0])
    gr = _sc_gr_kernel()
    y = None
    wflat = w.reshape(-1)
    for l in range(NL):
        R = gr(xs, src3, wflat)
        h2, stats = _agg_call(R, w, x, xd, snorm_n, preb[l],
                              P0[l], P12[l], P3[l], P4[l], Pb[l], postb[l],
                              N, 0)
        if l < NL - 1:
            x, xs, xd = _bnnext_call(h2, stats, bng[l], bnb[l],
                                     Ws[l + 1], Wd[l + 1])
        else:
            y = _bnout_call(h2, stats, bng[l], bnb[l],
                            r0W, r0b.reshape(1, -1), r1W, r1b.reshape(1, -1),
                            r2W, r2b.reshape(1, -1))
    return y
